# baseline (speedup 1.0000x reference)
import functools

import jax
import jax.numpy as jnp
from jax import lax
from jax.experimental import pallas as pl
from jax.experimental.pallas import tpu as pltpu
from jax.experimental.pallas import tpu_sc as plsc

N = 10000
E = 320000
D = 128
N_PAD = 10240
BLK = 512
NC = 2
NS = 16
NW = NC * NS
L = 16
E_PER_W = E // NW

_SC_PARAMS = pltpu.CompilerParams(needs_layout_passes=False)
_MESH = plsc.VectorSubcoreMesh(core_axis_name="c", subcore_axis_name="s")


def _make_edge_scatter(with_deg, CHUNK, NBUF):
  N_IT = E_PER_W // CHUNK
  out_type = [jax.ShapeDtypeStruct((NC * N_PAD, D), jnp.float32)]
  scratch = (
      [pltpu.VMEM((2, CHUNK), jnp.int32) for _ in range(2 * NBUF)]
      + [pltpu.VMEM((CHUNK, D), jnp.float32) for _ in range(NBUF)]
      + [pltpu.SemaphoreType.DMA for _ in range(2 * NBUF)]
      + [pltpu.SemaphoreType.DMA for _ in range(2 * NBUF)]
      + [pltpu.SemaphoreType.DMA]
  )
  if with_deg:
      out_type.append(jax.ShapeDtypeStruct((NW, N_PAD), jnp.float32))
      scratch.append(pltpu.VMEM((N_PAD,), jnp.float32))
  scratch.append(pltpu.VMEM_SHARED((N_PAD, D), jnp.float32))

  @functools.partial(pl.kernel, mesh=_MESH, out_type=out_type,
                     scratch_types=scratch, compiler_params=_SC_PARAMS)
  def _edge_scatter(*args):
    NI = 2 * NBUF
    if with_deg:
        sd_hbm, m_hbm, z2_hbm, z1_hbm, out_hbm, deg_hbm, *rest = args
    else:
        sd_hbm, m_hbm, z2_hbm, out_hbm, *rest = args
    idxb = rest[:NI]
    bufs = rest[NI:NI + NBUF]
    isems = rest[NI + NBUF:2 * NI + NBUF]
    gsems = rest[2 * NI + NBUF:2 * NI + 2 * NBUF]
    ssems = rest[2 * NI + 2 * NBUF:2 * NI + 3 * NBUF]
    if with_deg:
        zsem, hist_v, agg_s = rest[2 * NI + 3 * NBUF:]
    else:
        zsem, agg_s = rest[2 * NI + 3 * NBUF:]
    cid = lax.axis_index("c")
    sid = lax.axis_index("s")
    wid = cid * NS + sid
    rows_ps = N_PAD // NS
    r0 = sid * rows_ps

    def idx_load(chunk, i):
        pltpu.async_copy(sd_hbm.at[wid, chunk], idxb[i], isems[i])

    def idx_wait(chunk, i):
        pltpu.make_async_copy(sd_hbm.at[wid, chunk], idxb[i], isems[i]).wait()

    def gather(b, i):
        pltpu.async_copy(m_hbm.at[idxb[i].at[0]], bufs[b], gsems[b])

    def gather_wait(b, i):
        pltpu.make_async_copy(m_hbm.at[idxb[i].at[0]], bufs[b],
                              gsems[b]).wait()

    def scatter(b, i):
        pltpu.async_copy(bufs[b], agg_s.at[idxb[i].at[1]], ssems[b],
                         add=True)

    def scatter_wait(b, i):
        pltpu.make_async_copy(bufs[b], agg_s.at[idxb[i].at[1]],
                              ssems[b]).wait()

    zcp = pltpu.async_copy(z2_hbm.at[pl.ds(r0, rows_ps)],
                           agg_s.at[pl.ds(r0, rows_ps)], zsem)
    for b in range(NBUF):
        idx_load(b, b)
    for b in range(NBUF - 1):
        idx_wait(b, b)
        gather(b, b)
    if with_deg:
        pltpu.sync_copy(z1_hbm, hist_v)
    zcp.wait()
    plsc.subcore_barrier()

    ones16 = jnp.full((L,), 1.0, jnp.float32)
    tailmask = lax.iota(jnp.int32, L) >= (L - CHUNK % L) if CHUNK % L else None

    def hist_chunk(ib):
        for j in range(CHUNK // L):
            plsc.addupdate_scatter(hist_v, [idxb[ib][1, pl.ds(j * L, L)]],
                                   ones16)
        if CHUNK % L:
            plsc.addupdate_scatter(hist_v,
                                   [idxb[ib][1, pl.ds(CHUNK - L, L)]],
                                   ones16, mask=tailmask)

    def visit(k, b, ib):
        b2 = (b + NBUF - 1) % NBUF
        ib_g = (ib + NBUF - 1) % NI
        ib_l = (ib + NBUF) % NI

        @pl.when(k < N_IT)
        def _():
            gather_wait(b, ib)
            scatter(b, ib)
            if with_deg:
                hist_chunk(ib)

        @pl.when((k >= 1) & (k <= N_IT))
        def _():
            scatter_wait(b2, (ib + NI - 1) % NI)

        @pl.when(k + NBUF - 1 < N_IT)
        def _():
            idx_wait(k + NBUF - 1, ib_g)
            gather(b2, ib_g)

        @pl.when(k + NBUF < N_IT)
        def _():
            idx_load(k + NBUF, ib_l)

    def round_body(r, carry):
        for j in range(NI):
            k = r * NI + j
            visit(k, j % NBUF, j)
        return carry

    n_rounds = (N_IT + NI) // NI
    lax.fori_loop(0, n_rounds, round_body, 0)
    if with_deg:
        pltpu.sync_copy(hist_v, deg_hbm.at[wid])
    plsc.subcore_barrier()

    pltpu.sync_copy(agg_s.at[pl.ds(r0, rows_ps)],
                    out_hbm.at[pl.ds(cid * N_PAD + r0, rows_ps)])

  return _edge_scatter


_edge_scatter_l1 = _make_edge_scatter(True, 40, 3)
_edge_scatter_l2 = _make_edge_scatter(False, 80, 2)


def _pre1_body(h_ref, qt_ref, qb_ref, o_ref):
    m = jnp.dot(h_ref[...], qt_ref[...], preferred_element_type=jnp.float32)
    o_ref[...] = jnp.maximum(m + qb_ref[...], 0.0)


def _mid_body(h_ref, a0_ref, a1_ref, degp_ref, w1a_ref, w1b_ref, w1b_b_ref,
              q2t_ref, q2b_ref, h1_ref, m2_ref, deg_ref):
    deg = jnp.sum(degp_ref[...], axis=0)
    hn = (a0_ref[...] + a1_ref[...]) / jnp.maximum(deg, 1.0)[:, None]
    z = jnp.dot(h_ref[...], w1a_ref[...], preferred_element_type=jnp.float32)
    z = z + jnp.dot(hn, w1b_ref[...], preferred_element_type=jnp.float32)
    z = jnp.maximum(z + w1b_b_ref[...], 0.0)
    nrm = jnp.sqrt(jnp.sum(z * z, axis=1, keepdims=True))
    h1 = z / (nrm + 1e-6)
    h1_ref[...] = h1
    m2 = jnp.dot(h1, q2t_ref[...], preferred_element_type=jnp.float32)
    m2_ref[...] = jnp.maximum(m2 + q2b_ref[...], 0.0)
    deg_ref[...] = deg


def _post_body(h1_ref, a0_ref, a1_ref, deg_ref, w2a_ref, w2b_ref, w2b_b_ref,
               gt_ref, gb_ref, gs_ref, o_ref):
    deg = deg_ref[...]
    hn = (a0_ref[...] + a1_ref[...]) / jnp.maximum(deg, 1.0)[:, None]
    z = jnp.dot(h1_ref[...], w2a_ref[...], preferred_element_type=jnp.float32)
    z = z + jnp.dot(hn, w2b_ref[...], preferred_element_type=jnp.float32)
    z = jnp.maximum(z + w2b_b_ref[...], 0.0)
    nrm = jnp.sqrt(jnp.sum(z * z, axis=1, keepdims=True))
    h2 = z / (nrm + 1e-6)
    out = jnp.dot(h2, gt_ref[...], preferred_element_type=jnp.float32)
    o_ref[...] = gs_ref[...] * jnp.maximum(out + gb_ref[...], 0.0)


def _row_spec(width):
    return pl.BlockSpec((BLK, width), lambda i: (i, 0))


def _full_spec(shape):
    return pl.BlockSpec(shape, lambda i: tuple(0 for _ in shape))


_GRID = N_PAD // BLK
_vec_spec = pl.BlockSpec((BLK,), lambda i: (i,))

_pre1 = pl.pallas_call(
    _pre1_body,
    grid=(_GRID,),
    in_specs=[_row_spec(D), _full_spec((D, D)), _full_spec((1, D))],
    out_specs=_row_spec(D),
    out_shape=jax.ShapeDtypeStruct((N, D), jnp.float32),
)

_a1_spec = pl.BlockSpec((BLK, D), lambda i: (i + _GRID, 0))

_mid = pl.pallas_call(
    _mid_body,
    grid=(_GRID,),
    in_specs=[_row_spec(D), _row_spec(D), _a1_spec,
              pl.BlockSpec((NW, BLK), lambda i: (0, i)),
              _full_spec((D, D)), _full_spec((D, D)), _full_spec((1, D)),
              _full_spec((D, D)), _full_spec((1, D))],
    out_specs=[_row_spec(D), _row_spec(D), _vec_spec],
    out_shape=[jax.ShapeDtypeStruct((N, D), jnp.float32),
               jax.ShapeDtypeStruct((N, D), jnp.float32),
               jax.ShapeDtypeStruct((N,), jnp.float32)],
)

_post = pl.pallas_call(
    _post_body,
    grid=(_GRID,),
    in_specs=[_row_spec(D), _row_spec(D), _a1_spec, _vec_spec,
              _full_spec((D, D)), _full_spec((D, D)), _full_spec((1, D)),
              _full_spec((D, D)), _full_spec((1, D)), _full_spec((1, D))],
    out_specs=_row_spec(D),
    out_shape=jax.ShapeDtypeStruct((N, D), jnp.float32),
)


@jax.jit
def kernel(g, h, Q1_w, Q1_b, W1_w, W1_b, Q2_w, Q2_b, W2_w, W2_b, G_w, G_b,
           g_scale):
    gi = g.astype(jnp.int32)

    def mk_sd(chunk):
        n_it = E_PER_W // chunk
        return jnp.stack([gi[0].reshape(NW, n_it, chunk),
                          gi[1].reshape(NW, n_it, chunk)], axis=2)

    sd_a = mk_sd(40)
    sd_b = mk_sd(80)

    zeros_nd = jnp.zeros((N_PAD, D), jnp.float32)
    zeros_1d = jnp.zeros((N_PAD,), jnp.float32)

    m1 = _pre1(h, Q1_w.T, Q1_b[None, :])
    agg1, degp = _edge_scatter_l1(sd_a, m1, zeros_nd, zeros_1d)

    h1, m2, deg = _mid(h, agg1, agg1, degp,
                       W1_w[:, :D].T, W1_w[:, D:].T, W1_b[None, :],
                       Q2_w.T, Q2_b[None, :])

    agg2, = _edge_scatter_l2(sd_b, m2, zeros_nd)

    gs = jnp.broadcast_to(g_scale.astype(jnp.float32), (1, D))
    out = _post(h1, agg2, agg2, deg,
                W2_w[:, :D].T, W2_w[:, D:].T, W2_b[None, :],
                G_w.T, G_b[None, :], gs)
    return out


# speedup vs baseline: 1.0133x; 1.0133x over previous
import functools

import jax
import jax.numpy as jnp
from jax import lax
from jax.experimental import pallas as pl
from jax.experimental.pallas import tpu as pltpu
from jax.experimental.pallas import tpu_sc as plsc

N = 10000
E = 320000
D = 128
N_PAD = 10240
BLK = 512
NC = 2
NS = 16
NW = NC * NS
CHUNK = 40
NBUF = 3
L = 16
E_PER_W = E // NW
N_IT = E_PER_W // CHUNK

_SC_PARAMS = pltpu.CompilerParams(needs_layout_passes=False)
_MESH = plsc.VectorSubcoreMesh(core_axis_name="c", subcore_axis_name="s")


def _make_edge_scatter(with_deg):
  out_type = [jax.ShapeDtypeStruct((NC * N_PAD, D), jnp.float32)]
  scratch = (
      [pltpu.VMEM((2, CHUNK), jnp.int32) for _ in range(2 * NBUF)]
      + [pltpu.VMEM((CHUNK, D), jnp.float32) for _ in range(NBUF)]
      + [pltpu.SemaphoreType.DMA for _ in range(2 * NBUF)]
      + [pltpu.SemaphoreType.DMA for _ in range(2 * NBUF)]
      + [pltpu.SemaphoreType.DMA]
  )
  if with_deg:
      out_type.append(jax.ShapeDtypeStruct((NW, N_PAD), jnp.float32))
      scratch.append(pltpu.VMEM((N_PAD,), jnp.float32))
  scratch.append(pltpu.VMEM_SHARED((N_PAD, D), jnp.float32))

  @functools.partial(pl.kernel, mesh=_MESH, out_type=out_type,
                     scratch_types=scratch, compiler_params=_SC_PARAMS)
  def _edge_scatter(*args):
    NI = 2 * NBUF
    if with_deg:
        sd_hbm, m_hbm, z2_hbm, z1_hbm, out_hbm, deg_hbm, *rest = args
    else:
        sd_hbm, m_hbm, z2_hbm, out_hbm, *rest = args
    idxb = rest[:NI]
    bufs = rest[NI:NI + NBUF]
    isems = rest[NI + NBUF:2 * NI + NBUF]
    gsems = rest[2 * NI + NBUF:2 * NI + 2 * NBUF]
    ssems = rest[2 * NI + 2 * NBUF:2 * NI + 3 * NBUF]
    if with_deg:
        zsem, hist_v, agg_s = rest[2 * NI + 3 * NBUF:]
    else:
        zsem, agg_s = rest[2 * NI + 3 * NBUF:]
    cid = lax.axis_index("c")
    sid = lax.axis_index("s")
    wid = cid * NS + sid
    rows_ps = N_PAD // NS
    r0 = sid * rows_ps

    def idx_load(chunk, i):
        pltpu.async_copy(sd_hbm.at[wid, chunk], idxb[i], isems[i])

    def idx_wait(chunk, i):
        pltpu.make_async_copy(sd_hbm.at[wid, chunk], idxb[i], isems[i]).wait()

    def gather(b, i):
        pltpu.async_copy(m_hbm.at[idxb[i].at[0]], bufs[b], gsems[b])

    def gather_wait(b, i):
        pltpu.make_async_copy(m_hbm.at[idxb[i].at[0]], bufs[b],
                              gsems[b]).wait()

    def scatter(b, i):
        pltpu.async_copy(bufs[b], agg_s.at[idxb[i].at[1]], ssems[b],
                         add=True)

    def scatter_wait(b, i):
        pltpu.make_async_copy(bufs[b], agg_s.at[idxb[i].at[1]],
                              ssems[b]).wait()

    zcp = pltpu.async_copy(z2_hbm.at[pl.ds(r0, rows_ps)],
                           agg_s.at[pl.ds(r0, rows_ps)], zsem)
    for b in range(NBUF):
        idx_load(b, b)
    for b in range(NBUF - 1):
        idx_wait(b, b)
        gather(b, b)
    if with_deg:
        pltpu.sync_copy(z1_hbm, hist_v)
    zcp.wait()
    plsc.subcore_barrier()

    ones16 = jnp.full((L,), 1.0, jnp.float32)
    tailmask = lax.iota(jnp.int32, L) >= (L - CHUNK % L) if CHUNK % L else None

    def hist_chunk(ib):
        for j in range(CHUNK // L):
            plsc.addupdate_scatter(hist_v, [idxb[ib][1, pl.ds(j * L, L)]],
                                   ones16)
        if CHUNK % L:
            plsc.addupdate_scatter(hist_v,
                                   [idxb[ib][1, pl.ds(CHUNK - L, L)]],
                                   ones16, mask=tailmask)

    def visit(k, b, ib):
        b2 = (b + NBUF - 1) % NBUF
        ib_g = (ib + NBUF - 1) % NI
        ib_l = (ib + NBUF) % NI

        @pl.when(k < N_IT)
        def _():
            gather_wait(b, ib)
            scatter(b, ib)
            if with_deg:
                hist_chunk(ib)

        @pl.when((k >= 1) & (k <= N_IT))
        def _():
            scatter_wait(b2, (ib + NI - 1) % NI)

        @pl.when(k + NBUF - 1 < N_IT)
        def _():
            idx_wait(k + NBUF - 1, ib_g)
            gather(b2, ib_g)

        @pl.when(k + NBUF < N_IT)
        def _():
            idx_load(k + NBUF, ib_l)

    def round_body(r, carry):
        for j in range(NI):
            k = r * NI + j
            visit(k, j % NBUF, j)
        return carry

    n_rounds = (N_IT + NI) // NI
    lax.fori_loop(0, n_rounds, round_body, 0)
    if with_deg:
        pltpu.sync_copy(hist_v, deg_hbm.at[wid])
    plsc.subcore_barrier()

    pltpu.sync_copy(agg_s.at[pl.ds(r0, rows_ps)],
                    out_hbm.at[pl.ds(cid * N_PAD + r0, rows_ps)])

  return _edge_scatter


_edge_scatter_l1 = _make_edge_scatter(True)
_edge_scatter_l2 = _make_edge_scatter(False)


def _pre1_body(h_ref, qt_ref, qb_ref, o_ref):
    m = jnp.dot(h_ref[...], qt_ref[...], preferred_element_type=jnp.float32)
    o_ref[...] = jnp.maximum(m + qb_ref[...], 0.0)


def _mid_body(h_ref, a0_ref, a1_ref, degp_ref, w1a_ref, w1b_ref, w1b_b_ref,
              q2t_ref, q2b_ref, h1_ref, m2_ref, deg_ref):
    deg = jnp.sum(degp_ref[...], axis=0)
    hn = (a0_ref[...] + a1_ref[...]) / jnp.maximum(deg, 1.0)[:, None]
    z = jnp.dot(h_ref[...], w1a_ref[...], preferred_element_type=jnp.float32)
    z = z + jnp.dot(hn, w1b_ref[...], preferred_element_type=jnp.float32)
    z = jnp.maximum(z + w1b_b_ref[...], 0.0)
    nrm = jnp.sqrt(jnp.sum(z * z, axis=1, keepdims=True))
    h1 = z / (nrm + 1e-6)
    h1_ref[...] = h1
    m2 = jnp.dot(h1, q2t_ref[...], preferred_element_type=jnp.float32)
    m2_ref[...] = jnp.maximum(m2 + q2b_ref[...], 0.0)
    deg_ref[...] = deg


def _post_body(h1_ref, a0_ref, a1_ref, deg_ref, w2a_ref, w2b_ref, w2b_b_ref,
               gt_ref, gb_ref, gs_ref, o_ref):
    deg = deg_ref[...]
    hn = (a0_ref[...] + a1_ref[...]) / jnp.maximum(deg, 1.0)[:, None]
    z = jnp.dot(h1_ref[...], w2a_ref[...], preferred_element_type=jnp.float32)
    z = z + jnp.dot(hn, w2b_ref[...], preferred_element_type=jnp.float32)
    z = jnp.maximum(z + w2b_b_ref[...], 0.0)
    nrm = jnp.sqrt(jnp.sum(z * z, axis=1, keepdims=True))
    h2 = z / (nrm + 1e-6)
    out = jnp.dot(h2, gt_ref[...], preferred_element_type=jnp.float32)
    o_ref[...] = gs_ref[...] * jnp.maximum(out + gb_ref[...], 0.0)


def _row_spec(width):
    return pl.BlockSpec((BLK, width), lambda i: (i, 0))


def _full_spec(shape):
    return pl.BlockSpec(shape, lambda i: tuple(0 for _ in shape))


_GRID = N_PAD // BLK
_vec_spec = pl.BlockSpec((BLK,), lambda i: (i,))

_pre1 = pl.pallas_call(
    _pre1_body,
    grid=(_GRID,),
    in_specs=[_row_spec(D), _full_spec((D, D)), _full_spec((1, D))],
    out_specs=_row_spec(D),
    out_shape=jax.ShapeDtypeStruct((N, D), jnp.float32),
)

_a1_spec = pl.BlockSpec((BLK, D), lambda i: (i + _GRID, 0))

_mid = pl.pallas_call(
    _mid_body,
    grid=(_GRID,),
    in_specs=[_row_spec(D), _row_spec(D), _a1_spec,
              pl.BlockSpec((NW, BLK), lambda i: (0, i)),
              _full_spec((D, D)), _full_spec((D, D)), _full_spec((1, D)),
              _full_spec((D, D)), _full_spec((1, D))],
    out_specs=[_row_spec(D), _row_spec(D), _vec_spec],
    out_shape=[jax.ShapeDtypeStruct((N, D), jnp.float32),
               jax.ShapeDtypeStruct((N, D), jnp.float32),
               jax.ShapeDtypeStruct((N,), jnp.float32)],
)

_post = pl.pallas_call(
    _post_body,
    grid=(_GRID,),
    in_specs=[_row_spec(D), _row_spec(D), _a1_spec, _vec_spec,
              _full_spec((D, D)), _full_spec((D, D)), _full_spec((1, D)),
              _full_spec((D, D)), _full_spec((1, D)), _full_spec((1, D))],
    out_specs=_row_spec(D),
    out_shape=jax.ShapeDtypeStruct((N, D), jnp.float32),
)


@jax.jit
def kernel(g, h, Q1_w, Q1_b, W1_w, W1_b, Q2_w, Q2_b, W2_w, W2_b, G_w, G_b,
           g_scale):
    gi = g.astype(jnp.int32)
    sd4d = jnp.stack([gi[0].reshape(NW, N_IT, CHUNK),
                      gi[1].reshape(NW, N_IT, CHUNK)], axis=2)

    zeros_nd = jnp.zeros((N_PAD, D), jnp.float32)
    zeros_1d = jnp.zeros((N_PAD,), jnp.float32)

    m1 = _pre1(h, Q1_w.T, Q1_b[None, :])
    agg1, degp = _edge_scatter_l1(sd4d, m1, zeros_nd, zeros_1d)

    h1, m2, deg = _mid(h, agg1, agg1, degp,
                       W1_w[:, :D].T, W1_w[:, D:].T, W1_b[None, :],
                       Q2_w.T, Q2_b[None, :])

    agg2, = _edge_scatter_l2(sd4d, m2, zeros_nd)

    gs = jnp.broadcast_to(g_scale.astype(jnp.float32), (1, D))
    out = _post(h1, agg2, agg2, deg,
                W2_w[:, :D].T, W2_w[:, D:].T, W2_b[None, :],
                G_w.T, G_b[None, :], gs)
    return out
